# speedup vs baseline: 3.2370x; 3.2370x over previous
import functools

import jax
import jax.numpy as jnp
from jax import lax
from jax.experimental import pallas as pl
from jax.experimental.pallas import tpu as pltpu
from jax.experimental.pallas import tpu_sc as plsc

NUM_ROWS = 100000
DIM = 128
B_TOTAL = 16384 * 100

_info = plsc.get_sparse_core_info()
NC = _info.num_cores
NS = _info.num_subcores
NW = NC * NS
B_PER_W = B_TOTAL // NW
CHUNK = 128
N_CHUNKS = B_PER_W // CHUNK


def _make_gather():
    mesh = plsc.VectorSubcoreMesh(core_axis_name="c", subcore_axis_name="s")

    @functools.partial(
        pl.kernel,
        mesh=mesh,
        out_type=jax.ShapeDtypeStruct((B_TOTAL, DIM), jnp.float32),
        scratch_types=[
            pltpu.VMEM((CHUNK,), jnp.int32),
            pltpu.VMEM((CHUNK, DIM), jnp.float32),
            pltpu.SemaphoreType.DMA,
        ],
    )
    def gather_kernel(idx_hbm, table_hbm, out_hbm, idx_v, rows_v, sem):
        wid = lax.axis_index("s") * NC + lax.axis_index("c")
        base = wid * B_PER_W

        def body(g, carry):
            off = base + g * CHUNK
            pltpu.sync_copy(idx_hbm.at[pl.ds(off, CHUNK)], idx_v)
            pltpu.async_copy(table_hbm.at[idx_v], rows_v, sem).wait()
            pltpu.sync_copy(rows_v, out_hbm.at[pl.ds(off, CHUNK)])
            return carry

        lax.fori_loop(0, N_CHUNKS, body, 0)

    return gather_kernel


_gather = _make_gather()


@jax.jit
def kernel(indices, embedding):
    flat_idx = indices.reshape(-1)
    out = _gather(flat_idx, embedding)
    return out.reshape(indices.shape + (DIM,))


# speedup vs baseline: 4.0505x; 1.2513x over previous
import functools

import jax
import jax.numpy as jnp
from jax import lax
from jax.experimental import pallas as pl
from jax.experimental.pallas import tpu as pltpu
from jax.experimental.pallas import tpu_sc as plsc

NUM_ROWS = 100000
DIM = 128
B_TOTAL = 16384 * 100

_info = plsc.get_sparse_core_info()
NC = _info.num_cores
NS = _info.num_subcores
NW = NC * NS
B_PER_W = B_TOTAL // NW
CHUNK = 128
N_CHUNKS = B_PER_W // CHUNK
NBUF = 4
LOOKAHEAD = 2


def _make_gather():
    mesh = plsc.VectorSubcoreMesh(core_axis_name="c", subcore_axis_name="s")

    @functools.partial(
        pl.kernel,
        mesh=mesh,
        out_type=jax.ShapeDtypeStruct((B_TOTAL, DIM), jnp.float32),
        scratch_types=[
            pltpu.VMEM((N_CHUNKS, CHUNK), jnp.int32),
            pltpu.VMEM((NBUF, CHUNK, DIM), jnp.float32),
            pltpu.SemaphoreType.DMA,
            pltpu.SemaphoreType.DMA,
        ],
    )
    def gather_kernel(idx_hbm, table_hbm, out_hbm, idx_v, rows_v, gsem, wsem):
        wid = lax.axis_index("s") * NC + lax.axis_index("c")
        base = wid * B_PER_W

        pltpu.sync_copy(idx_hbm.at[wid], idx_v)

        def start_gather(g):
            pltpu.async_copy(table_hbm.at[idx_v.at[g]], rows_v.at[g % NBUF], gsem)

        def wait_gather(g):
            pltpu.make_async_copy(
                table_hbm.at[idx_v.at[g]], rows_v.at[g % NBUF], gsem
            ).wait()

        def start_write(g):
            pltpu.async_copy(
                rows_v.at[g % NBUF], out_hbm.at[pl.ds(base + g * CHUNK, CHUNK)], wsem
            )

        def wait_write(g):
            pltpu.make_async_copy(
                rows_v.at[g % NBUF], out_hbm.at[pl.ds(base + g * CHUNK, CHUNK)], wsem
            ).wait()

        for g in range(LOOKAHEAD):
            start_gather(g)

        def body(g, carry):
            @pl.when(g + LOOKAHEAD < N_CHUNKS)
            def _():
                start_gather(g + LOOKAHEAD)

            wait_gather(g)
            start_write(g)

            @pl.when(g >= 1)
            def _():
                wait_write(g - 1)

            return carry

        lax.fori_loop(0, N_CHUNKS, body, 0)
        wait_write(N_CHUNKS - 1)

    return gather_kernel


_gather = _make_gather()


@jax.jit
def kernel(indices, embedding):
    idx = indices.reshape(NW, N_CHUNKS, CHUNK)
    out = _gather(idx, embedding)
    return out.reshape(indices.shape + (DIM,))


# speedup vs baseline: 7.1353x; 1.7616x over previous
import functools

import jax
import jax.numpy as jnp
from jax import lax
from jax.experimental import pallas as pl
from jax.experimental.pallas import tpu as pltpu
from jax.experimental.pallas import tpu_sc as plsc

NUM_ROWS = 100000
DIM = 128
N_OUTER = 16384
N_INNER = 100

_info = plsc.get_sparse_core_info()
NC = _info.num_cores
NS = _info.num_subcores
NW = NC * NS
ROWS_PER_W = N_OUTER // NW
NBUF = 4
LOOKAHEAD = 2


def _make_gather():
    mesh = plsc.VectorSubcoreMesh(core_axis_name="c", subcore_axis_name="s")

    @functools.partial(
        pl.kernel,
        mesh=mesh,
        out_type=jax.ShapeDtypeStruct((N_OUTER, N_INNER, DIM), jnp.float32),
        scratch_types=[
            pltpu.VMEM((ROWS_PER_W, N_INNER), jnp.int32),
            pltpu.VMEM((NBUF, N_INNER, DIM), jnp.float32),
            pltpu.SemaphoreType.DMA,
            pltpu.SemaphoreType.DMA,
        ],
    )
    def gather_kernel(idx_hbm, table_hbm, out_hbm, idx_v, rows_v, gsem, wsem):
        wid = lax.axis_index("s") * NC + lax.axis_index("c")
        base = wid * ROWS_PER_W

        pltpu.sync_copy(idx_hbm.at[wid], idx_v)

        def start_gather(g):
            pltpu.async_copy(table_hbm.at[idx_v.at[g]], rows_v.at[g % NBUF], gsem)

        def wait_gather(g):
            pltpu.make_async_copy(
                table_hbm.at[idx_v.at[g]], rows_v.at[g % NBUF], gsem
            ).wait()

        def start_write(g):
            pltpu.async_copy(rows_v.at[g % NBUF], out_hbm.at[base + g], wsem)

        def wait_write(g):
            pltpu.make_async_copy(
                rows_v.at[g % NBUF], out_hbm.at[base + g], wsem
            ).wait()

        for g in range(LOOKAHEAD):
            start_gather(g)

        def body(g, carry):
            @pl.when(g + LOOKAHEAD < ROWS_PER_W)
            def _():
                start_gather(g + LOOKAHEAD)

            wait_gather(g)
            start_write(g)

            @pl.when(g >= 1)
            def _():
                wait_write(g - 1)

            return carry

        lax.fori_loop(0, ROWS_PER_W, body, 0)
        wait_write(ROWS_PER_W - 1)

    return gather_kernel


_gather = _make_gather()


@jax.jit
def kernel(indices, embedding):
    idx = indices.reshape(NW, ROWS_PER_W, N_INNER)
    return _gather(idx, embedding)


# speedup vs baseline: 7.1471x; 1.0017x over previous
import functools

import jax
import jax.numpy as jnp
from jax import lax
from jax.experimental import pallas as pl
from jax.experimental.pallas import tpu as pltpu
from jax.experimental.pallas import tpu_sc as plsc

NUM_ROWS = 100000
DIM = 128
N_OUTER = 16384
N_INNER = 100

_info = plsc.get_sparse_core_info()
NC = _info.num_cores
NS = _info.num_subcores
NW = NC * NS
ROWS_PER_W = N_OUTER // NW
NBUF = 4
LOOKAHEAD = 2


def _make_gather():
    mesh = plsc.VectorSubcoreMesh(core_axis_name="c", subcore_axis_name="s")

    @functools.partial(
        pl.kernel,
        mesh=mesh,
        out_type=jax.ShapeDtypeStruct((N_OUTER, N_INNER, DIM), jnp.float32),
        compiler_params=pltpu.CompilerParams(use_tc_tiling_on_sc=True),
        scratch_types=[
            pltpu.VMEM((ROWS_PER_W, N_INNER), jnp.int32),
            pltpu.VMEM((NBUF, N_INNER, DIM), jnp.float32),
            pltpu.SemaphoreType.DMA,
            pltpu.SemaphoreType.DMA,
        ],
    )
    def gather_kernel(idx_hbm, table_hbm, out_hbm, idx_v, rows_v, gsem, wsem):
        wid = lax.axis_index("s") * NC + lax.axis_index("c")
        base = wid * ROWS_PER_W

        pltpu.sync_copy(idx_hbm.at[wid], idx_v)

        def start_gather(g):
            pltpu.async_copy(table_hbm.at[idx_v.at[g]], rows_v.at[g % NBUF], gsem)

        def wait_gather(g):
            pltpu.make_async_copy(
                table_hbm.at[idx_v.at[g]], rows_v.at[g % NBUF], gsem
            ).wait()

        def start_write(g):
            pltpu.async_copy(rows_v.at[g % NBUF], out_hbm.at[base + g], wsem)

        def wait_write(g):
            pltpu.make_async_copy(
                rows_v.at[g % NBUF], out_hbm.at[base + g], wsem
            ).wait()

        for g in range(LOOKAHEAD):
            start_gather(g)

        def body(g, carry):
            @pl.when(g + LOOKAHEAD < ROWS_PER_W)
            def _():
                start_gather(g + LOOKAHEAD)

            wait_gather(g)
            start_write(g)

            @pl.when(g >= 1)
            def _():
                wait_write(g - 1)

            return carry

        lax.fori_loop(0, ROWS_PER_W, body, 0)
        wait_write(ROWS_PER_W - 1)

    return gather_kernel


_gather = _make_gather()


@jax.jit
def kernel(indices, embedding):
    idx = indices.reshape(NW, ROWS_PER_W, N_INNER)
    return _gather(idx, embedding)


# speedup vs baseline: 13.1691x; 1.8426x over previous
import functools

import jax
import jax.numpy as jnp
from jax import lax
from jax.experimental import pallas as pl
from jax.experimental.pallas import tpu as pltpu
from jax.experimental.pallas import tpu_sc as plsc

NUM_ROWS = 100000
DIM = 128
N_OUTER = 16384
N_INNER = 100
B_TOTAL = N_OUTER * N_INNER

_info = plsc.get_sparse_core_info()
NC = _info.num_cores
NS = _info.num_subcores
NW = NC * NS
OUTER_PER_W = N_OUTER // NW
B_PER_W = B_TOTAL // NW
CHUNK = 128
CHUNKS_PER_RUN = OUTER_PER_W // CHUNK
N_CHUNKS = B_PER_W // CHUNK
NBUF = 4
LOOKAHEAD = 2


def _make_gather():
    mesh = plsc.VectorSubcoreMesh(core_axis_name="c", subcore_axis_name="s")

    @functools.partial(
        pl.kernel,
        mesh=mesh,
        out_type=jax.ShapeDtypeStruct((B_TOTAL, DIM), jnp.float32),
        scratch_types=[
            pltpu.VMEM((B_PER_W,), jnp.int32),
            pltpu.VMEM((NBUF, CHUNK, DIM), jnp.float32),
            pltpu.SemaphoreType.DMA,
            pltpu.SemaphoreType.DMA,
        ],
    )
    def gather_kernel(idx_hbm, table_hbm, out_hbm, idx_v, rows_v, gsem, wsem):
        wid = lax.axis_index("s") * NC + lax.axis_index("c")

        pltpu.sync_copy(idx_hbm.at[wid], idx_v)

        def out_slice(g):
            j = g // CHUNKS_PER_RUN
            c = g % CHUNKS_PER_RUN
            row = j * N_OUTER + wid * OUTER_PER_W + c * CHUNK
            return out_hbm.at[pl.ds(row, CHUNK)]

        def start_gather(g):
            pltpu.async_copy(
                table_hbm.at[idx_v.at[pl.ds(g * CHUNK, CHUNK)]],
                rows_v.at[g % NBUF],
                gsem,
            )

        def wait_gather(g):
            pltpu.make_async_copy(
                table_hbm.at[idx_v.at[pl.ds(g * CHUNK, CHUNK)]],
                rows_v.at[g % NBUF],
                gsem,
            ).wait()

        def start_write(g):
            pltpu.async_copy(rows_v.at[g % NBUF], out_slice(g), wsem)

        def wait_write(g):
            pltpu.make_async_copy(rows_v.at[g % NBUF], out_slice(g), wsem).wait()

        for g in range(LOOKAHEAD):
            start_gather(g)

        def body(g, carry):
            @pl.when(g + LOOKAHEAD < N_CHUNKS)
            def _():
                start_gather(g + LOOKAHEAD)

            wait_gather(g)
            start_write(g)

            @pl.when(g >= 1)
            def _():
                wait_write(g - 1)

            return carry

        lax.fori_loop(0, N_CHUNKS, body, 0)
        wait_write(N_CHUNKS - 1)

    return gather_kernel


_gather = _make_gather()


@jax.jit
def kernel(indices, embedding):
    idx = indices.reshape(NW, OUTER_PER_W, N_INNER).transpose(0, 2, 1)
    idx = idx.reshape(NW, B_PER_W)
    out = _gather(idx, embedding)
    return out.reshape(N_INNER, N_OUTER, DIM).transpose(1, 0, 2)
